# Initial kernel scaffold; baseline (speedup 1.0000x reference)
import jax
import jax.numpy as jnp
from jax.experimental import pallas as pl


def kernel(nodes, edges, src, params, connections, node_mask, lengths):
    raise NotImplementedError("write your pallas kernel here")


# speedup vs baseline: 8.5668x; 8.5668x over previous
import functools

import jax
import jax.numpy as jnp
from jax import lax
from jax.experimental import pallas as pl
from jax.experimental.pallas import tpu as pltpu
from jax.experimental.pallas import tpu_sc as plsc

B, L, K = 4, 512, 30
NF, EF, H, NL = 10, 11, 256, 30
FF = H * 4
N = B * L
E = N * K
EPS = 1e-05

BLK = 128
GRID = N // BLK

SC_WORKERS = 32
ROWS_W = E // SC_WORKERS
CHUNK = 128
NCHUNK = ROWS_W // CHUNK


def _ln(x, g, b):
    m = jnp.mean(x, axis=-1, keepdims=True)
    d = x - m
    v = jnp.mean(d * d, axis=-1, keepdims=True)
    return d * lax.rsqrt(v + EPS) * g + b


def _sc_gather_kernel(idx_hbm, table_hbm, out_hbm, idx_v, rows_a, rows_b,
                      gsem, wsem_a, wsem_b):
    wid = lax.axis_index("s") * 2 + lax.axis_index("c")
    base = wid * ROWS_W
    pltpu.sync_copy(idx_hbm.at[pl.ds(base, ROWS_W)], idx_v)
    bufs = (rows_a, rows_b)
    wsems = (wsem_a, wsem_b)
    wr = [None, None]
    for j in range(NCHUNK):
        s = j % 2
        if wr[s] is not None:
            wr[s].wait()
        pltpu.async_copy(
            table_hbm.at[idx_v.at[pl.ds(j * CHUNK, CHUNK)]], bufs[s],
            gsem).wait()
        wr[s] = pltpu.async_copy(
            bufs[s], out_hbm.at[pl.ds(base + j * CHUNK, CHUNK)], wsems[s])
    for s in range(2):
        if wr[s] is not None:
            wr[s].wait()


def _sc_gather(idx, table):
    mesh = plsc.VectorSubcoreMesh(core_axis_name="c", subcore_axis_name="s")
    fn = pl.kernel(
        _sc_gather_kernel,
        out_type=jax.ShapeDtypeStruct((E, H), jnp.float32),
        mesh=mesh,
        scratch_types=[
            pltpu.VMEM((ROWS_W,), jnp.int32),
            pltpu.VMEM((CHUNK, H), jnp.float32),
            pltpu.VMEM((CHUNK, H), jnp.float32),
            pltpu.SemaphoreType.DMA,
            pltpu.SemaphoreType.DMA,
            pltpu.SemaphoreType.DMA,
        ],
    )
    return fn(idx, table)


def _init_kernel(nodes_ref, src_ref, wvn_ref, wvs_ref, bv_ref, w1n_ref,
                 hv_ref, n_ref):
    hv = (jnp.dot(nodes_ref[...], wvn_ref[...],
                  preferred_element_type=jnp.float32)
          + jnp.dot(src_ref[...], wvs_ref[...],
                    preferred_element_type=jnp.float32)
          + bv_ref[...])
    hv_ref[...] = hv
    n_ref[...] = jnp.dot(hv, w1n_ref[...], preferred_element_type=jnp.float32)


def _layer_kernel(hv_ref, g_ref, e_ref, w1v_ref, w1e_ref, we_ref, be1_ref,
                  w2_ref, b2_ref, w3_ref, b3_ref, ln1g_ref, ln1b_ref,
                  wf1_ref, bf1_ref, wf2_ref, bf2_ref, ln2g_ref, ln2b_ref,
                  w1nn_ref, hv_out_ref, n_out_ref):
    hv = hv_ref[...]
    wce = jnp.dot(we_ref[...], w1e_ref[...],
                  preferred_element_type=jnp.float32)
    base = (jnp.dot(hv, w1v_ref[...], preferred_element_type=jnp.float32)
            + be1_ref[...])
    base_t = jnp.broadcast_to(base[None], (K, BLK, H)).reshape(K * BLK, H)
    ek = e_ref[...].reshape(K * BLK, EF)
    ew = jnp.dot(ek, wce, preferred_element_type=jnp.float32)
    gk = g_ref[...].reshape(K * BLK, H)
    m1 = jnp.maximum(base_t + ew + gk, 0.0)
    m2 = jnp.maximum(
        jnp.dot(m1, w2_ref[...], preferred_element_type=jnp.float32)
        + b2_ref[...], 0.0)
    s = jnp.sum(m2.reshape(K, BLK, H), axis=0)
    upd = (jnp.dot(s, w3_ref[...], preferred_element_type=jnp.float32)
           * (1.0 / 30.0) + b3_ref[...])
    h1 = _ln(hv + upd, ln1g_ref[...], ln1b_ref[...])
    dh = (jnp.dot(
        jnp.maximum(
            jnp.dot(h1, wf1_ref[...], preferred_element_type=jnp.float32)
            + bf1_ref[...], 0.0),
        wf2_ref[...], preferred_element_type=jnp.float32) + bf2_ref[...])
    h2 = _ln(h1 + dh, ln2g_ref[...], ln2b_ref[...])
    hv_out_ref[...] = h2
    n_out_ref[...] = jnp.dot(h2, w1nn_ref[...],
                             preferred_element_type=jnp.float32)


def _head1_kernel(hv_ref, pw1_ref, pb1_ref, plg_ref, plb_ref, pw2_ref,
                  pb2_ref, out_ref):
    h = jnp.maximum(
        jnp.dot(hv_ref[...], pw1_ref[...], preferred_element_type=jnp.float32)
        + pb1_ref[...], 0.0)
    h = _ln(h, plg_ref[...], plb_ref[...])
    s = (jnp.dot(h, pw2_ref[...], preferred_element_type=jnp.float32)
         + pb2_ref[...])
    out_ref[...] = jnp.maximum(s, 0.0)


def _head2_kernel(r_ref, l1g_ref, l1b_ref, rw1_ref, rb1_ref, l2g_ref,
                  l2b_ref, rw2_ref, rb2_ref, out_ref):
    r = _ln(r_ref[...], l1g_ref[...], l1b_ref[...])
    m = jnp.maximum(
        jnp.dot(r, rw1_ref[...], preferred_element_type=jnp.float32)
        + rb1_ref[...], 0.0)
    m = _ln(m, l2g_ref[...], l2b_ref[...])
    out_ref[...] = (jnp.dot(m, rw2_ref[...],
                            preferred_element_type=jnp.float32)
                    + rb2_ref[...] + 0.5)


def _row(v):
    return v.reshape(1, -1)


def kernel(nodes, edges, src, params, connections, node_mask, lengths):
    p = params
    nodes_f = nodes.reshape(N, NF)
    src_f = src.reshape(N, NL)
    e_km = edges.reshape(N, K, EF).transpose(1, 0, 2)
    gidx = (connections.astype(jnp.int32)
            + (jnp.arange(B, dtype=jnp.int32) * L)[:, None, None])
    gidx_km = gidx.reshape(N, K).T.reshape(E)

    layers = p['layers']
    w1n = [lp['w1'][2 * H:] for lp in layers]

    full = lambda shape: pl.BlockSpec(shape, lambda *_: (0,) * len(shape))

    hv, ntab = pl.pallas_call(
        _init_kernel,
        grid=(1,),
        in_specs=[full((N, NF)), full((N, NL)), full((NF, H)), full((NL, H)),
                  full((1, H)), full((H, H))],
        out_specs=[full((N, H)), full((N, H))],
        out_shape=[jax.ShapeDtypeStruct((N, H), jnp.float32),
                   jax.ShapeDtypeStruct((N, H), jnp.float32)],
    )(nodes_f, src_f, p['wv'][:NF], p['wv'][NF:], _row(p['bv']), w1n[0])

    blk = lambda shape: pl.BlockSpec(shape, lambda i: (i,) + (0,) * (len(shape) - 1))
    km_blk = lambda shape: pl.BlockSpec(shape, lambda i: (0, i, 0))

    for t, lp in enumerate(layers):
        g = _sc_gather(gidx_km, ntab).reshape(K, N, H)
        w1nn = w1n[t + 1] if t + 1 < len(layers) else w1n[t]
        hv, ntab = pl.pallas_call(
            _layer_kernel,
            grid=(GRID,),
            in_specs=[blk((BLK, H)), km_blk((K, BLK, H)), km_blk((K, BLK, EF)),
                      full((H, H)), full((H, H)), full((EF, H)), full((1, H)),
                      full((H, H)), full((1, H)), full((H, H)), full((1, H)),
                      full((1, H)), full((1, H)),
                      full((H, FF)), full((1, FF)), full((FF, H)), full((1, H)),
                      full((1, H)), full((1, H)), full((H, H))],
            out_specs=[blk((BLK, H)), blk((BLK, H))],
            out_shape=[jax.ShapeDtypeStruct((N, H), jnp.float32),
                       jax.ShapeDtypeStruct((N, H), jnp.float32)],
            compiler_params=pltpu.CompilerParams(
                dimension_semantics=("arbitrary",)),
        )(hv, g, e_km,
          lp['w1'][:H], lp['w1'][H:2 * H], p['we'], _row(p['be'] @ lp['w1'][H:2 * H] + lp['b1']),
          lp['w2'], _row(lp['b2']), lp['w3'], _row(lp['b3']),
          _row(lp['ln1g']), _row(lp['ln1b']),
          lp['wf1'], _row(lp['bf1']), lp['wf2'], _row(lp['bf2']),
          _row(lp['ln2g']), _row(lp['ln2b']), w1nn)

    r = pl.pallas_call(
        _head1_kernel,
        grid=(1,),
        in_specs=[full((N, H)), full((H, 2 * H)), full((1, 2 * H)),
                  full((1, 2 * H)), full((1, 2 * H)), full((2 * H, 1)),
                  full((1, 1))],
        out_specs=[full((N, 1))],
        out_shape=[jax.ShapeDtypeStruct((N, 1), jnp.float32)],
    )(hv, p['phi_w1'], _row(p['phi_b1']), _row(p['phi_lng']),
      _row(p['phi_lnb']), p['phi_w2'], _row(p['phi_b2']))[0]

    out = pl.pallas_call(
        _head2_kernel,
        grid=(1,),
        in_specs=[full((B, L)), full((1, L)), full((1, L)), full((L, 2 * L)),
                  full((1, 2 * L)), full((1, 2 * L)), full((1, 2 * L)),
                  full((2 * L, 1)), full((1, 1))],
        out_specs=[full((B, 1))],
        out_shape=[jax.ShapeDtypeStruct((B, 1), jnp.float32)],
    )(r.reshape(B, L), _row(p['rho_ln1g']), _row(p['rho_ln1b']),
      p['rho_w1'], _row(p['rho_b1']), _row(p['rho_ln2g']),
      _row(p['rho_ln2b']), p['rho_w2'], _row(p['rho_b2']))[0]

    return out.reshape(B)
